# baseline (speedup 1.0000x reference)
import dataclasses
import functools

import jax
import jax.numpy as jnp
from jax import lax
from jax.experimental import pallas as pl
from jax.experimental.pallas import tpu as pltpu
from jax.experimental.pallas import tpu_sc as plsc

GS = 128
NLEV = 16
MAXQ = 15.0
L = 16
NC = 2
NS = 16
NW = NC * NS
CH = 8192
NB = 2
GPI = 2
INV_MAXQ2 = 2.0 / 15.0

SC_GROUPS = 8192
TCR = 256


def _sc_quantize(x1, lookup_values):
    n = x1.size
    per_w = n // NW
    nch = per_w // CH
    mesh = plsc.VectorSubcoreMesh(core_axis_name="c", subcore_axis_name="s")
    cp = pltpu.CompilerParams()
    if "needs_layout_passes" in pltpu.CompilerParams.__dataclass_fields__:
        cp = dataclasses.replace(cp, needs_layout_passes=False)

    @functools.partial(
        pl.kernel,
        mesh=mesh,
        out_type=jax.ShapeDtypeStruct((n,), jnp.float32),
        scratch_types=[
            pltpu.VMEM((NLEV,), jnp.float32),
            pltpu.VMEM((CH,), jnp.float32),
            pltpu.VMEM((CH,), jnp.float32),
            pltpu.VMEM((CH,), jnp.float32),
            pltpu.VMEM((CH,), jnp.float32),
            pltpu.SemaphoreType.DMA((NB,)),
            pltpu.SemaphoreType.DMA((NB,)),
        ],
        compiler_params=cp,
    )
    def run(x_hbm, lut_hbm, o_hbm, lut_vmem, ibuf0, ibuf1, obuf0, obuf1,
            isem, osem):
        ibufs = [ibuf0, ibuf1]
        obufs = [obuf0, obuf1]
        pltpu.sync_copy(lut_hbm, lut_vmem)
        lutv = lut_vmem[...]

        def take(v, idx):
            return v.at[idx].get(mode="promise_in_bounds")

        lane = lax.iota(jnp.int32, L)
        shifted = take(lutv, jnp.minimum(lane + 1, NLEV - 1))
        midsv = (lutv + shifted) * 0.5

        i7 = jnp.full((L,), 7, jnp.int32)
        s8 = jnp.full((L,), 8, jnp.int32)
        s4 = jnp.full((L,), 4, jnp.int32)
        s2 = jnp.full((L,), 2, jnp.int32)
        s1 = jnp.full((L,), 1, jnp.int32)
        s0 = jnp.zeros((L,), jnp.int32)

        def search(q, msv, valv):
            lo = jnp.where(q > take(msv, i7), s8, s0)
            lo = lo + jnp.where(q > take(msv, lo + 3), s4, s0)
            lo = lo + jnp.where(q > take(msv, lo + 1), s2, s0)
            lo = lo + jnp.where(q > take(msv, lo), s1, s0)
            return take(valv, lo)

        zv = search(jnp.full((L,), 8.0, jnp.float32), midsv, lutv)
        mz = midsv - zv
        oz = lutv - zv

        wid = lax.axis_index("s") * NC + lax.axis_index("c")
        base = wid * per_w

        def in_cp(i, b):
            return pltpu.make_async_copy(
                x_hbm.at[pl.ds(base + i * CH, CH)], ibufs[b], isem.at[b]
            )

        def out_cp(i, b):
            return pltpu.make_async_copy(
                obufs[b], o_hbm.at[pl.ds(base + i * CH, CH)], osem.at[b]
            )

        for b in range(NB):
            in_cp(b, b).start()

        @pl.loop(0, nch, step=NB)
        def _(i0):
            for b in range(NB):
                i = i0 + b
                in_cp(i, b).wait()

                @pl.when(i >= NB)
                def _():
                    out_cp(i - NB, b).wait()

                @pl.loop(0, CH // GS, step=GPI)
                def _(g0):
                    for gg in range(GPI):
                        gbase = (g0 + gg) * GS
                        xs = [
                            ibufs[b][pl.ds(gbase + j * L, L)]
                            for j in range(GS // L)
                        ]
                        avs = [jnp.abs(v) for v in xs]
                        while len(avs) > 1:
                            avs = [
                                jnp.maximum(avs[k], avs[k + 1])
                                for k in range(0, len(avs) - 1, 2)
                            ] + ([avs[-1]] if len(avs) % 2 else [])
                        amax = jnp.max(avs[0])
                        amaxv = jnp.full((L,), amax, jnp.float32)
                        scale = jnp.where(
                            amaxv == 0.0, INV_MAXQ2, amaxv * INV_MAXQ2
                        )
                        msv = mz * scale
                        outv = oz * scale
                        for j in range(GS // L):
                            obufs[b][pl.ds(gbase + j * L, L)] = search(
                                xs[j], msv, outv
                            )

                @pl.when(i + NB < nch)
                def _():
                    in_cp(i + NB, b).start()

                out_cp(i, b).start()

        for b in range(NB):
            out_cp(nch - NB + b, b).wait()

    return run(x1, lookup_values)


def _tc_body(mz_ref, oz_ref, x_ref, o_ref):
    xb = x_ref[...]
    amax = jnp.max(jnp.abs(xb), axis=1, keepdims=True)
    scale = jnp.where(amax == 0.0, INV_MAXQ2, amax * INV_MAXQ2)
    out = jnp.broadcast_to(oz_ref[0, 0] * scale, xb.shape)
    for k in range(1, NLEV):
        out = jnp.where(xb > mz_ref[0, k - 1] * scale, oz_ref[0, k] * scale,
                        out)
    o_ref[...] = out


def _tc_quantize(xg, mz, oz):
    rows = xg.shape[0]
    return pl.pallas_call(
        _tc_body,
        out_shape=jax.ShapeDtypeStruct(xg.shape, jnp.float32),
        grid=(rows // TCR,),
        in_specs=[
            pl.BlockSpec(memory_space=pltpu.SMEM),
            pl.BlockSpec(memory_space=pltpu.SMEM),
            pl.BlockSpec((TCR, GS), lambda i: (i, 0)),
        ],
        out_specs=pl.BlockSpec((TCR, GS), lambda i: (i, 0)),
    )(mz, oz, xg)


def kernel(x, lookup_values):
    shape = x.shape
    g2 = x.reshape(-1, GS)
    n_sc = SC_GROUPS * GS

    lut = lookup_values
    mids = (lut[:-1] + lut[1:]) * 0.5
    zero = lut[jnp.argmin(jnp.abs(lut - (MAXQ + 1.0) / 2.0))]
    mz = (mids - zero).reshape(1, NLEV - 1)
    oz = (lut - zero).reshape(1, NLEV)

    sc_out = _sc_quantize(g2[:SC_GROUPS].reshape(n_sc), lookup_values)
    tc_out = _tc_quantize(g2[SC_GROUPS:], mz, oz)
    out = jnp.concatenate([sc_out.reshape(-1, GS), tc_out], axis=0)
    return out.reshape(shape)


# speedup vs baseline: 1.0305x; 1.0305x over previous
import dataclasses
import functools

import jax
import jax.numpy as jnp
from jax import lax
from jax.experimental import pallas as pl
from jax.experimental.pallas import tpu as pltpu
from jax.experimental.pallas import tpu_sc as plsc

GS = 128
NLEV = 16
MAXQ = 15.0
L = 16
NC = 2
NS = 16
NW = NC * NS
CH = 8192
NB = 2
GPI = 2
INV_MAXQ2 = 2.0 / 15.0

SC_GROUPS = 8192
TCR = 256


def _sc_quantize(x1, lookup_values):
    n = x1.size
    per_w = n // NW
    nch = per_w // CH
    mesh = plsc.VectorSubcoreMesh(core_axis_name="c", subcore_axis_name="s")
    cp = pltpu.CompilerParams()
    if "needs_layout_passes" in pltpu.CompilerParams.__dataclass_fields__:
        cp = dataclasses.replace(cp, needs_layout_passes=False)

    @functools.partial(
        pl.kernel,
        mesh=mesh,
        out_type=jax.ShapeDtypeStruct((n,), jnp.float32),
        scratch_types=[
            pltpu.VMEM((NLEV,), jnp.float32),
            pltpu.VMEM((CH,), jnp.float32),
            pltpu.VMEM((CH,), jnp.float32),
            pltpu.VMEM((CH,), jnp.float32),
            pltpu.VMEM((CH,), jnp.float32),
            pltpu.SemaphoreType.DMA((NB,)),
            pltpu.SemaphoreType.DMA((NB,)),
        ],
        compiler_params=cp,
    )
    def run(x_hbm, lut_hbm, o_hbm, lut_vmem, ibuf0, ibuf1, obuf0, obuf1,
            isem, osem):
        ibufs = [ibuf0, ibuf1]
        obufs = [obuf0, obuf1]
        pltpu.sync_copy(lut_hbm, lut_vmem)
        lutv = lut_vmem[...]

        def take(v, idx):
            return v.at[idx].get(mode="promise_in_bounds")

        lane = lax.iota(jnp.int32, L)
        shifted = take(lutv, jnp.minimum(lane + 1, NLEV - 1))
        midsv = (lutv + shifted) * 0.5

        i7 = jnp.full((L,), 7, jnp.int32)
        s8 = jnp.full((L,), 8, jnp.int32)
        s4 = jnp.full((L,), 4, jnp.int32)
        s2 = jnp.full((L,), 2, jnp.int32)
        s1 = jnp.full((L,), 1, jnp.int32)
        s0 = jnp.zeros((L,), jnp.int32)

        def search(q, msv, valv):
            lo = jnp.where(q > take(msv, i7), s8, s0)
            lo = lo + jnp.where(q > take(msv, lo + 3), s4, s0)
            lo = lo + jnp.where(q > take(msv, lo + 1), s2, s0)
            lo = lo + jnp.where(q > take(msv, lo), s1, s0)
            return take(valv, lo)

        zv = search(jnp.full((L,), 8.0, jnp.float32), midsv, lutv)
        mz = midsv - zv
        oz = lutv - zv

        wid = lax.axis_index("s") * NC + lax.axis_index("c")
        base = wid * per_w

        def in_cp(i, b):
            return pltpu.make_async_copy(
                x_hbm.at[pl.ds(base + i * CH, CH)], ibufs[b], isem.at[b]
            )

        def out_cp(i, b):
            return pltpu.make_async_copy(
                obufs[b], o_hbm.at[pl.ds(base + i * CH, CH)], osem.at[b]
            )

        for b in range(NB):
            in_cp(b, b).start()

        @pl.loop(0, nch, step=NB)
        def _(i0):
            for b in range(NB):
                i = i0 + b
                in_cp(i, b).wait()

                @pl.when(i >= NB)
                def _():
                    out_cp(i - NB, b).wait()

                @pl.loop(0, CH // GS, step=GPI)
                def _(g0):
                    for gg in range(GPI):
                        gbase = (g0 + gg) * GS
                        xs = [
                            ibufs[b][pl.ds(gbase + j * L, L)]
                            for j in range(GS // L)
                        ]
                        avs = [jnp.abs(v) for v in xs]
                        while len(avs) > 1:
                            avs = [
                                jnp.maximum(avs[k], avs[k + 1])
                                for k in range(0, len(avs) - 1, 2)
                            ] + ([avs[-1]] if len(avs) % 2 else [])
                        amax = jnp.max(avs[0])
                        amaxv = jnp.full((L,), amax, jnp.float32)
                        scale = jnp.where(
                            amaxv == 0.0, INV_MAXQ2, amaxv * INV_MAXQ2
                        )
                        msv = mz * scale
                        outv = oz * scale
                        for j in range(GS // L):
                            obufs[b][pl.ds(gbase + j * L, L)] = search(
                                xs[j], msv, outv
                            )

                @pl.when(i + NB < nch)
                def _():
                    in_cp(i + NB, b).start()

                out_cp(i, b).start()

        for b in range(NB):
            out_cp(nch - NB + b, b).wait()

    return run(x1, lookup_values)


def _tc_body(mz_ref, oz_ref, x_ref, o_ref):
    xb = x_ref[...]
    amax = jnp.max(jnp.abs(xb), axis=1, keepdims=True)
    scale = jnp.where(amax == 0.0, INV_MAXQ2, amax * INV_MAXQ2)
    q = xb * (1.0 / scale)
    out = jnp.full_like(xb, oz_ref[0, 0])
    for k in range(1, NLEV):
        out = jnp.where(q > mz_ref[0, k - 1], oz_ref[0, k], out)
    o_ref[...] = out * scale


def _tc_quantize(xg, mz, oz):
    rows = xg.shape[0]
    return pl.pallas_call(
        _tc_body,
        out_shape=jax.ShapeDtypeStruct(xg.shape, jnp.float32),
        grid=(rows // TCR,),
        in_specs=[
            pl.BlockSpec(memory_space=pltpu.SMEM),
            pl.BlockSpec(memory_space=pltpu.SMEM),
            pl.BlockSpec((TCR, GS), lambda i: (i, 0)),
        ],
        out_specs=pl.BlockSpec((TCR, GS), lambda i: (i, 0)),
    )(mz, oz, xg)


def kernel(x, lookup_values):
    shape = x.shape
    g2 = x.reshape(-1, GS)
    n_sc = SC_GROUPS * GS

    lut = lookup_values
    mids = (lut[:-1] + lut[1:]) * 0.5
    zero = lut[jnp.argmin(jnp.abs(lut - (MAXQ + 1.0) / 2.0))]
    mz = (mids - zero).reshape(1, NLEV - 1)
    oz = (lut - zero).reshape(1, NLEV)

    sc_out = _sc_quantize(g2[:SC_GROUPS].reshape(n_sc), lookup_values)
    tc_out = _tc_quantize(g2[SC_GROUPS:], mz, oz)
    out = jnp.concatenate([sc_out.reshape(-1, GS), tc_out], axis=0)
    return out.reshape(shape)


# speedup vs baseline: 1.7212x; 1.6702x over previous
import dataclasses
import functools

import jax
import jax.numpy as jnp
from jax import lax
from jax.experimental import pallas as pl
from jax.experimental.pallas import tpu as pltpu
from jax.experimental.pallas import tpu_sc as plsc

GS = 128
NLEV = 16
L = 16
NC = 2
NS = 16
NW = NC * NS
CH = 8192
NB = 2
GPI = 2
INV_MAXQ2 = 2.0 / 15.0


def kernel(x, lookup_values):
    shape = x.shape
    n = x.size
    x1 = x.reshape(n)
    per_w = n // NW
    nch = per_w // CH
    mesh = plsc.VectorSubcoreMesh(core_axis_name="c", subcore_axis_name="s")
    cp = pltpu.CompilerParams()
    if "needs_layout_passes" in pltpu.CompilerParams.__dataclass_fields__:
        cp = dataclasses.replace(cp, needs_layout_passes=False)

    @functools.partial(
        pl.kernel,
        mesh=mesh,
        out_type=jax.ShapeDtypeStruct((n,), jnp.float32),
        scratch_types=[
            pltpu.VMEM((NLEV,), jnp.float32),
            pltpu.VMEM((CH,), jnp.float32),
            pltpu.VMEM((CH,), jnp.float32),
            pltpu.VMEM((CH,), jnp.float32),
            pltpu.VMEM((CH,), jnp.float32),
            pltpu.SemaphoreType.DMA((NB,)),
            pltpu.SemaphoreType.DMA((NB,)),
        ],
        compiler_params=cp,
    )
    def run(x_hbm, lut_hbm, o_hbm, lut_vmem, ibuf0, ibuf1, obuf0, obuf1, isem, osem):
        ibufs = [ibuf0, ibuf1]
        obufs = [obuf0, obuf1]
        pltpu.sync_copy(lut_hbm, lut_vmem)
        lutv = lut_vmem[...]

        def take(v, idx):
            return v.at[idx].get(mode="promise_in_bounds")

        lane = lax.iota(jnp.int32, L)
        shifted = take(lutv, jnp.minimum(lane + 1, NLEV - 1))
        midsv = (lutv + shifted) * 0.5

        i7 = jnp.full((L,), 7, jnp.int32)
        s8 = jnp.full((L,), 8, jnp.int32)
        s4 = jnp.full((L,), 4, jnp.int32)
        s2 = jnp.full((L,), 2, jnp.int32)
        s1 = jnp.full((L,), 1, jnp.int32)
        s0 = jnp.zeros((L,), jnp.int32)

        def search(q, msv, valv):
            lo = jnp.where(q > take(msv, i7), s8, s0)
            lo = lo + jnp.where(q > take(msv, lo + 3), s4, s0)
            lo = lo + jnp.where(q > take(msv, lo + 1), s2, s0)
            lo = lo + jnp.where(q > take(msv, lo), s1, s0)
            return take(valv, lo)

        zv = search(jnp.full((L,), 8.0, jnp.float32), midsv, lutv)
        mz = midsv - zv
        oz = lutv - zv

        wid = lax.axis_index("s") * NC + lax.axis_index("c")
        base = wid * per_w

        def in_cp(i, b):
            return pltpu.make_async_copy(
                x_hbm.at[pl.ds(base + i * CH, CH)], ibufs[b], isem.at[b]
            )

        def out_cp(i, b):
            return pltpu.make_async_copy(
                obufs[b], o_hbm.at[pl.ds(base + i * CH, CH)], osem.at[b]
            )

        for b in range(NB):
            in_cp(b, b).start()

        @pl.loop(0, nch, step=NB)
        def _(i0):
            for b in range(NB):
                i = i0 + b
                in_cp(i, b).wait()

                @pl.when(i >= NB)
                def _():
                    out_cp(i - NB, b).wait()

                @pl.loop(0, CH // GS, step=GPI)
                def _(g0):
                    for gg in range(GPI):
                        gbase = (g0 + gg) * GS
                        xs = [
                            ibufs[b][pl.ds(gbase + j * L, L)]
                            for j in range(GS // L)
                        ]
                        avs = [jnp.abs(v) for v in xs]
                        while len(avs) > 1:
                            avs = [
                                jnp.maximum(avs[k], avs[k + 1])
                                for k in range(0, len(avs) - 1, 2)
                            ] + ([avs[-1]] if len(avs) % 2 else [])
                        amax = jnp.max(avs[0])
                        amaxv = jnp.full((L,), amax, jnp.float32)
                        scale = jnp.where(
                            amaxv == 0.0, INV_MAXQ2, amaxv * INV_MAXQ2
                        )
                        msv = mz * scale
                        outv = oz * scale
                        for j in range(GS // L):
                            obufs[b][pl.ds(gbase + j * L, L)] = search(
                                xs[j], msv, outv
                            )

                @pl.when(i + NB < nch)
                def _():
                    in_cp(i + NB, b).start()

                out_cp(i, b).start()

        for b in range(NB):
            out_cp(nch - NB + b, b).wait()

    return run(x1, lookup_values).reshape(shape)
